# baseline (speedup 1.0000x reference)
import functools

import jax
import jax.numpy as jnp
from jax import lax
from jax.experimental import pallas as pl
from jax.experimental.pallas import tpu as pltpu
from jax.experimental.pallas import tpu_sc as plsc

EPS = 1e-6


def _make_sc_gather(num_rows, hidden):
    info = plsc.get_sparse_core_info()
    nc = info.num_cores

    rows_per_w = 8
    num_workers = num_rows // rows_per_w

    mesh = plsc.VectorSubcoreMesh(core_axis_name="c", subcore_axis_name="s")

    @functools.partial(
        pl.kernel,
        mesh=mesh,
        out_type=jax.ShapeDtypeStruct((num_rows, hidden), jnp.float32),
        scratch_types=[
            pltpu.VMEM((rows_per_w,), jnp.int32),
            pltpu.VMEM((rows_per_w, hidden), jnp.float32),
            pltpu.SemaphoreType.DMA,
        ],
    )
    def gather(idx_hbm, table_hbm, out_hbm, idx_v, rows_v, sem):
        wid = lax.axis_index("s") * nc + lax.axis_index("c")

        @pl.when(wid < num_workers)
        def _():
            base = wid * rows_per_w
            pltpu.sync_copy(idx_hbm.at[pl.ds(base, rows_per_w)], idx_v)
            pltpu.async_copy(table_hbm.at[idx_v], rows_v, sem).wait()
            pltpu.sync_copy(rows_v, out_hbm.at[pl.ds(base, rows_per_w)])

    return gather


def _add_ln_block(feat_ref, ovec_ref, gamma_ref, beta_ref, out_ref):
    x = feat_ref[...] + ovec_ref[...]
    mean = jnp.mean(x, axis=-1, keepdims=True)
    meansq = jnp.mean(x * x, axis=-1, keepdims=True)
    var = meansq - mean * mean
    inv = lax.rsqrt(var + EPS)
    out_ref[...] = (x - mean) * (inv * gamma_ref[...]) + beta_ref[...]


def _add_ln(feats, ovecs3, gamma2, beta2, b_blk):
    b, p, h = feats.shape
    grid = (b // b_blk,)
    return pl.pallas_call(
        _add_ln_block,
        grid=grid,
        in_specs=[
            pl.BlockSpec((b_blk, p, h), lambda i: (i, 0, 0)),
            pl.BlockSpec((b_blk, 1, h), lambda i: (i, 0, 0)),
            pl.BlockSpec((1, h), lambda i: (0, 0)),
            pl.BlockSpec((1, h), lambda i: (0, 0)),
        ],
        out_specs=pl.BlockSpec((b_blk, p, h), lambda i: (i, 0, 0)),
        out_shape=jax.ShapeDtypeStruct((b, p, h), jnp.float32),
        compiler_params=pltpu.CompilerParams(
            dimension_semantics=("parallel",),
            vmem_limit_bytes=100 * 1024 * 1024,
        ),
    )(feats, ovecs3, gamma2, beta2)


def kernel(image_features, image_positions, order_emb, ln_gamma, ln_beta):
    b, p, h = image_features.shape
    num_rows = image_positions.shape[0]
    ovecs = _make_sc_gather(num_rows, h)(image_positions, order_emb)
    ovecs3 = ovecs.reshape(num_rows, 1, h)
    gamma2 = ln_gamma.reshape(1, h)
    beta2 = ln_beta.reshape(1, h)
    return _add_ln(image_features, ovecs3, gamma2, beta2, b_blk=4)


# speedup vs baseline: 1.0224x; 1.0224x over previous
import functools

import jax
import jax.numpy as jnp
from jax import lax
from jax.experimental import pallas as pl
from jax.experimental.pallas import tpu as pltpu
from jax.experimental.pallas import tpu_sc as plsc

EPS = 1e-6
NBUF = 4


def _make_sc_gather(num_rows, hidden):
    info = plsc.get_sparse_core_info()
    nc = info.num_cores

    rows_per_w = 8
    num_workers = num_rows // rows_per_w

    mesh = plsc.VectorSubcoreMesh(core_axis_name="c", subcore_axis_name="s")

    @functools.partial(
        pl.kernel,
        mesh=mesh,
        out_type=jax.ShapeDtypeStruct((num_rows, hidden), jnp.float32),
        scratch_types=[
            pltpu.VMEM((rows_per_w,), jnp.int32),
            pltpu.VMEM((rows_per_w, hidden), jnp.float32),
            pltpu.SemaphoreType.DMA,
        ],
    )
    def gather(idx_hbm, table_hbm, out_hbm, idx_v, rows_v, sem):
        wid = lax.axis_index("s") * nc + lax.axis_index("c")

        @pl.when(wid < num_workers)
        def _():
            base = wid * rows_per_w
            pltpu.sync_copy(idx_hbm.at[pl.ds(base, rows_per_w)], idx_v)
            pltpu.async_copy(table_hbm.at[idx_v], rows_v, sem).wait()
            pltpu.sync_copy(rows_v, out_hbm.at[pl.ds(base, rows_per_w)])

    return gather


def _make_add_ln(b, p, h):
    assert b % NBUF == 0
    num_g = b // NBUF

    def body(feat_hbm, ovec_v, gamma_v, beta_v, out_hbm, inbuf, outbuf,
             insem, outsem):
        def start_in(img, slot):
            pltpu.make_async_copy(
                feat_hbm.at[img], inbuf.at[slot], insem.at[slot]).start()

        def start_out(img, slot):
            pltpu.make_async_copy(
                outbuf.at[slot], out_hbm.at[img], outsem.at[slot]).start()

        for s in range(NBUF):
            start_in(s, s)

        def g_body(g, carry):
            for s in range(NBUF):
                img = g * NBUF + s
                pltpu.make_async_copy(
                    feat_hbm.at[img], inbuf.at[s], insem.at[s]).wait()

                @pl.when(g > 0)
                def _():
                    pltpu.make_async_copy(
                        outbuf.at[s], out_hbm.at[img - NBUF],
                        outsem.at[s]).wait()

                x = inbuf[s] + ovec_v[img]
                mean = jnp.mean(x, axis=-1, keepdims=True)
                meansq = jnp.mean(x * x, axis=-1, keepdims=True)
                inv = lax.rsqrt(meansq - mean * mean + EPS)
                outbuf[s] = (x - mean) * (inv * gamma_v[...]) + beta_v[...]

                start_out(img, s)

                @pl.when(g < num_g - 1)
                def _():
                    start_in(img + NBUF, s)
            return carry

        lax.fori_loop(0, num_g, g_body, 0)

        for s in range(NBUF):
            pltpu.make_async_copy(
                outbuf.at[s], out_hbm.at[b - NBUF + s], outsem.at[s]).wait()

    return pl.pallas_call(
        body,
        in_specs=[
            pl.BlockSpec(memory_space=pl.ANY),
            pl.BlockSpec(memory_space=pltpu.VMEM),
            pl.BlockSpec(memory_space=pltpu.VMEM),
            pl.BlockSpec(memory_space=pltpu.VMEM),
        ],
        out_specs=pl.BlockSpec(memory_space=pl.ANY),
        out_shape=jax.ShapeDtypeStruct((b, p, h), jnp.float32),
        scratch_shapes=[
            pltpu.VMEM((NBUF, p, h), jnp.float32),
            pltpu.VMEM((NBUF, p, h), jnp.float32),
            pltpu.SemaphoreType.DMA((NBUF,)),
            pltpu.SemaphoreType.DMA((NBUF,)),
        ],
        compiler_params=pltpu.CompilerParams(
            vmem_limit_bytes=60 * 1024 * 1024,
        ),
    )


def kernel(image_features, image_positions, order_emb, ln_gamma, ln_beta):
    b, p, h = image_features.shape
    num_rows = image_positions.shape[0]
    ovecs = _make_sc_gather(num_rows, h)(image_positions, order_emb)
    ovecs3 = ovecs.reshape(num_rows, 1, h)
    gamma2 = ln_gamma.reshape(1, h)
    beta2 = ln_beta.reshape(1, h)
    return _make_add_ln(b, p, h)(image_features, ovecs3, gamma2, beta2)
